# baseline (speedup 1.0000x reference)
import jax
import jax.numpy as jnp
from jax import lax
from jax.experimental import pallas as pl
from jax.experimental.pallas import tpu as pltpu
from jax.experimental.pallas import tpu_sc as plsc

NC = 2
NS = 16
NW = NC * NS

BATCH = 4096
HIST = 200
VOCAB = 1000
D = 64
N_IDX = BATCH * HIST
B_PER_W = N_IDX // NW

G_ROWS = 256
N_GROUPS = B_PER_W // G_ROWS
BLK = 16
N_BLK = G_ROWS // BLK


def _body(tokens_hbm, table_hbm, out_hbm, tbl_v, idx_v, rows_v, wsem):
    wid = lax.axis_index("s") * NC + lax.axis_index("c")
    base = wid * B_PER_W
    pltpu.sync_copy(table_hbm, tbl_v)
    pltpu.sync_copy(tokens_hbm.at[pl.ds(base, B_PER_W)], idx_v)

    def compute(g, half):
        @pl.loop(0, N_BLK)
        def _blk(blk):
            src = g * G_ROWS + blk * BLK
            dst = (half * G_ROWS + blk * BLK) * D
            toks = idx_v[pl.ds(src, BLK)]
            for j in range(BLK):
                row = toks[j] * D
                o = dst + j * D
                for k in range(0, D, 16):
                    rows_v[pl.ds(o + k, 16)] = tbl_v[pl.ds(row + k, 16)]

    def write(g, half):
        return pltpu.make_async_copy(
            rows_v.at[pl.ds(half * G_ROWS * D, G_ROWS * D)],
            out_hbm.at[pl.ds((base + g * G_ROWS) * D, G_ROWS * D)],
            wsem.at[half],
        )

    compute(0, 0)
    write(0, 0).start()
    compute(1, 1)
    write(1, 1).start()

    @pl.loop(0, (N_GROUPS - 2) // 2)
    def _pair(p):
        g = 2 * p + 2
        write(g - 2, 0).wait()
        compute(g, 0)
        write(g, 0).start()
        write(g - 1, 1).wait()
        compute(g + 1, 1)
        write(g + 1, 1).start()

    write(N_GROUPS - 2, 0).wait()
    write(N_GROUPS - 1, 1).wait()


def kernel(tokens, table):
    flat = tokens.reshape(N_IDX)
    tbl_flat = table.reshape(VOCAB * D)
    mesh = plsc.VectorSubcoreMesh(core_axis_name="c", subcore_axis_name="s")
    out = pl.kernel(
        _body,
        out_type=jax.ShapeDtypeStruct((N_IDX * D,), jnp.float32),
        mesh=mesh,
        compiler_params=pltpu.CompilerParams(use_tc_tiling_on_sc=False),
        scratch_types=[
            pltpu.VMEM((VOCAB * D,), jnp.float32),
            pltpu.VMEM((B_PER_W,), jnp.int32),
            pltpu.VMEM((2 * G_ROWS * D,), jnp.float32),
            pltpu.SemaphoreType.DMA((2,)),
        ],
    )(flat, tbl_flat)
    return out.reshape(BATCH, HIST, D)
